# baseline (speedup 1.0000x reference)
import jax
import jax.numpy as jnp
from jax.experimental import pallas as pl
from jax.experimental.pallas import tpu as pltpu

BLOCK_V = 2048


def _body(params_ref, x_ref, m_ref, o_ref):
    inv_tau = 1.0 / params_ref[0]
    xs = (x_ref[...] * inv_tau).astype(jnp.bfloat16)
    mb = m_ref[...].astype(jnp.bfloat16)
    o_ref[...] = jax.lax.dot_general(
        mb, xs,
        dimension_numbers=(((1,), (1,)), ((), ())),
        preferred_element_type=jnp.float32,
    )


def kernel(x, y, memory, params):
    b, d = x.shape
    v = memory.shape[0]
    nb = pl.cdiv(v, BLOCK_V)
    out_t = pl.pallas_call(
        _body,
        grid=(nb,),
        in_specs=[
            pl.BlockSpec(memory_space=pltpu.SMEM),
            pl.BlockSpec((b, d), lambda i: (0, 0)),
            pl.BlockSpec((BLOCK_V, d), lambda i: (i, 0)),
        ],
        out_specs=pl.BlockSpec((BLOCK_V, b), lambda i: (i, 0)),
        out_shape=jax.ShapeDtypeStruct((v, b), jnp.float32),
        compiler_params=pltpu.CompilerParams(
            dimension_semantics=("arbitrary",),
        ),
    )(params, x, memory)
    return out_t.T


# speedup vs baseline: 1.0228x; 1.0228x over previous
import jax
import jax.numpy as jnp
from jax.experimental import pallas as pl
from jax.experimental.pallas import tpu as pltpu

BLOCK_V = 5120


def _body(params_ref, x_ref, m_ref, o_ref):
    inv_tau = 1.0 / params_ref[0]
    xs = (x_ref[...] * inv_tau).astype(jnp.bfloat16)
    mb = m_ref[...].astype(jnp.bfloat16)
    o_ref[...] = jax.lax.dot_general(
        mb, xs,
        dimension_numbers=(((1,), (1,)), ((), ())),
        preferred_element_type=jnp.float32,
    )


def kernel(x, y, memory, params):
    b, d = x.shape
    v = memory.shape[0]
    nb = pl.cdiv(v, BLOCK_V)
    out_t = pl.pallas_call(
        _body,
        grid=(nb,),
        in_specs=[
            pl.BlockSpec(memory_space=pltpu.SMEM),
            pl.BlockSpec((b, d), lambda i: (0, 0)),
            pl.BlockSpec((BLOCK_V, d), lambda i: (i, 0)),
        ],
        out_specs=pl.BlockSpec((BLOCK_V, b), lambda i: (i, 0)),
        out_shape=jax.ShapeDtypeStruct((v, b), jnp.float32),
        compiler_params=pltpu.CompilerParams(
            dimension_semantics=("arbitrary",),
        ),
    )(params, x, memory)
    return out_t.T


# speedup vs baseline: 1.0241x; 1.0013x over previous
import jax
import jax.numpy as jnp
from jax.experimental import pallas as pl
from jax.experimental.pallas import tpu as pltpu

BLOCK_V = 6272


def _body(params_ref, x_ref, m_ref, o_ref):
    inv_tau = 1.0 / params_ref[0]
    xs = (x_ref[...] * inv_tau).astype(jnp.bfloat16)
    mb = m_ref[...].astype(jnp.bfloat16)
    o_ref[...] = jax.lax.dot_general(
        mb, xs,
        dimension_numbers=(((1,), (1,)), ((), ())),
        preferred_element_type=jnp.float32,
    )


def kernel(x, y, memory, params):
    b, d = x.shape
    v = memory.shape[0]
    nb = pl.cdiv(v, BLOCK_V)
    out_t = pl.pallas_call(
        _body,
        grid=(nb,),
        in_specs=[
            pl.BlockSpec(memory_space=pltpu.SMEM),
            pl.BlockSpec((b, d), lambda i: (0, 0)),
            pl.BlockSpec((BLOCK_V, d), lambda i: (i, 0)),
        ],
        out_specs=pl.BlockSpec((BLOCK_V, b), lambda i: (i, 0)),
        out_shape=jax.ShapeDtypeStruct((v, b), jnp.float32),
        compiler_params=pltpu.CompilerParams(
            dimension_semantics=("arbitrary",),
        ),
    )(params, x, memory)
    return out_t.T
